# baseline (speedup 1.0000x reference)
import functools

import jax
import jax.numpy as jnp
from jax import lax
from jax.experimental import pallas as pl
from jax.experimental.pallas import tpu as pltpu
from jax.experimental.pallas import tpu_sc as plsc

NC, NS = 2, 16
NW = NC * NS
BATCH = 16384
DIM = 16
NF = 26
ROWS_PER_W = BATCH // NW
CHUNK = 128
NCH = ROWS_PER_W // CHUNK

_mesh = plsc.VectorSubcoreMesh(core_axis_name="c", subcore_axis_name="s")


@functools.partial(
    pl.kernel,
    out_type=jax.ShapeDtypeStruct((BATCH // CHUNK, CHUNK, NF * DIM), jnp.float32),
    mesh=_mesh,
    scratch_types=[
        pltpu.VMEM((NCH, CHUNK), jnp.int32),
        pltpu.VMEM((CHUNK, DIM), jnp.float32),
        pltpu.SemaphoreType.DMA,
    ],
    compiler_params=pltpu.CompilerParams(use_tc_tiling_on_sc=False),
)
def _embed_sc(*refs):
    idx_refs = refs[:NF]
    tab_refs = refs[NF:2 * NF]
    out_ref = refs[2 * NF]
    idx_v, rows_v, sem = refs[2 * NF + 1:]
    wid = lax.axis_index("s") * NC + lax.axis_index("c")
    r0 = wid * NCH
    for f in range(NF):
        pltpu.sync_copy(idx_refs[f].at[pl.ds(r0, NCH)], idx_v)

        @pl.loop(0, NCH)
        def _chunk(c, f=f):
            pltpu.async_copy(tab_refs[f].at[idx_v.at[c]], rows_v, sem).wait()
            pltpu.sync_copy(rows_v,
                            out_ref.at[r0 + c, :, pl.ds(f * DIM, DIM)])


def kernel(f00, f01, f02, f03, f04, f05, f06, f07, f08, f09, f10, f11, f12,
           f13, f14, f15, f16, f17, f18, f19, f20, f21, f22, f23, f24, f25,
           W_f00, W_f01, W_f02, W_f03, W_f04, W_f05, W_f06, W_f07, W_f08,
           W_f09, W_f10, W_f11, W_f12, W_f13, W_f14, W_f15, W_f16, W_f17,
           W_f18, W_f19, W_f20, W_f21, W_f22, W_f23, W_f24, W_f25):
    idx = [a.reshape(BATCH // CHUNK, CHUNK) for a in
           (f00, f01, f02, f03, f04, f05, f06, f07, f08, f09, f10, f11, f12,
            f13, f14, f15, f16, f17, f18, f19, f20, f21, f22, f23, f24, f25)]
    tabs = (W_f00, W_f01, W_f02, W_f03, W_f04, W_f05, W_f06, W_f07, W_f08,
            W_f09, W_f10, W_f11, W_f12, W_f13, W_f14, W_f15, W_f16, W_f17,
            W_f18, W_f19, W_f20, W_f21, W_f22, W_f23, W_f24, W_f25)
    out = _embed_sc(*idx, *tabs)
    return out.reshape(BATCH, NF * DIM)
